# baseline (speedup 1.0000x reference)
import functools

import jax
import jax.numpy as jnp
from jax import lax
from jax.experimental import pallas as pl
from jax.experimental.pallas import tpu as pltpu
from jax.experimental.pallas import tpu_sc as plsc

N = 10000
D = 128
E = 320000
EPS = 1e-5
NC = 2
NS = 16
NW = NC * NS
CH = 64
NCHUNK = E // CH
NGRP = NCHUNK // 8
CBASE = 152
CEXTRA = 8
NACC = 10240
RPT = NACC // NS


def _tile_range(w):
    g0 = (w * NGRP) // NW
    g1 = ((w + 1) * NGRP) // NW
    base = pl.multiple_of(g0 * 8, 8)
    has_extra = (g1 - g0) > 19
    return base, has_extra


def _mesh():
    return plsc.VectorSubcoreMesh(core_axis_name="c", subcore_axis_name="s")


def _deg_partials(dst_p):
    @functools.partial(
        pl.kernel,
        out_type=jax.ShapeDtypeStruct((NC, NACC), jnp.float32),
        mesh=_mesh(),
        scratch_types=[
            pltpu.VMEM((CBASE + CEXTRA, CH), jnp.int32),
            pltpu.VMEM((CH,), jnp.float32),
            pltpu.VMEM((RPT,), jnp.float32),
            pltpu.VMEM_SHARED((NACC,), jnp.float32),
            pltpu.SemaphoreType.DMA,
        ],
    )
    def k(dst_hbm, out_hbm, idx_v, ones_v, zeros_v, deg_sh, semd):
        c = lax.axis_index("c")
        s = lax.axis_index("s")
        w = s * NC + c
        base, has_extra = _tile_range(w)

        def fill_zeros(i, _):
            zeros_v[pl.ds(i * 16, 16)] = jnp.zeros((16,), jnp.float32)
            return 0

        lax.fori_loop(0, RPT // 16, fill_zeros, 0)

        def fill_ones(i, _):
            ones_v[pl.ds(i * 16, 16)] = jnp.ones((16,), jnp.float32)
            return 0

        lax.fori_loop(0, CH // 16, fill_ones, 0)

        pltpu.sync_copy(zeros_v, deg_sh.at[pl.ds(s * RPT, RPT)])
        plsc.subcore_barrier()

        pltpu.sync_copy(dst_hbm.at[pl.ds(base, CBASE)],
                        idx_v.at[pl.ds(0, CBASE)])

        @pl.when(has_extra)
        def _():
            pltpu.sync_copy(dst_hbm.at[pl.ds(base + CBASE, CEXTRA)],
                            idx_v.at[pl.ds(CBASE, CEXTRA)])

        def body(j, _):
            pltpu.async_copy(ones_v, deg_sh.at[idx_v.at[j]], semd, add=True)
            return 0

        lax.fori_loop(0, CBASE, body, 0)

        def drain(j, _):
            pltpu.make_async_copy(ones_v, deg_sh.at[idx_v.at[0]], semd).wait()
            return 0

        @pl.when(has_extra)
        def _():
            lax.fori_loop(CBASE, CBASE + CEXTRA, body, 0)
            lax.fori_loop(0, CEXTRA, drain, 0)

        lax.fori_loop(0, CBASE, drain, 0)
        plsc.subcore_barrier()
        pltpu.sync_copy(deg_sh.at[pl.ds(s * RPT, RPT)],
                        out_hbm.at[c, pl.ds(s * RPT, RPT)])

    return k(dst_p)


NBUF = 4
SEGS = (56, 56, 40)
SEGMAX = max(SEGS)


def _scatter_partials(hs, src_p, dst_p):
    @functools.partial(
        pl.kernel,
        out_type=jax.ShapeDtypeStruct((NC, NACC, D), jnp.float32),
        mesh=_mesh(),
        scratch_types=[
            pltpu.VMEM((SEGMAX, CH), jnp.int32),
            pltpu.VMEM((SEGMAX, CH), jnp.int32),
            [pltpu.VMEM((CH, D), jnp.float32)] * NBUF,
            pltpu.VMEM_SHARED((NACC, D), jnp.float32),
            [pltpu.SemaphoreType.DMA] * NBUF,
            [pltpu.SemaphoreType.DMA] * NBUF,
        ],
    )
    def k(hs_hbm, src_hbm, dst_hbm, out_hbm, src_v, dst_v, rows,
          acc_sh, semg, sems):
        c = lax.axis_index("c")
        s = lax.axis_index("s")
        w = s * NC + c
        tbase, has_extra = _tile_range(w)

        def fill_zeros(t, _):
            rows[0][t // 8, pl.ds((t % 8) * 16, 16)] = jnp.zeros(
                (16,), jnp.float32)
            return 0

        lax.fori_loop(0, CH * 8, fill_zeros, 0)

        def zero_acc(i, _):
            pltpu.sync_copy(rows[0], acc_sh.at[pl.ds(s * RPT + i * CH, CH)])
            return 0

        lax.fori_loop(0, RPT // CH, zero_acc, 0)
        plsc.subcore_barrier()

        def gather(j, b):
            pltpu.async_copy(hs_hbm.at[src_v.at[j]], rows[b], semg[b])

        def gwait(j, b):
            pltpu.make_async_copy(hs_hbm.at[src_v.at[j]], rows[b],
                                  semg[b]).wait()

        def scat(j, b):
            pltpu.async_copy(rows[b], acc_sh.at[dst_v.at[j]], sems[b],
                             add=True)

        def swait(j, b):
            pltpu.make_async_copy(rows[b], acc_sh.at[dst_v.at[j]],
                                  sems[b]).wait()

        def run_seg(seg_base, nch):
            pltpu.sync_copy(src_hbm.at[pl.ds(seg_base, nch)],
                            src_v.at[pl.ds(0, nch)])
            pltpu.sync_copy(dst_hbm.at[pl.ds(seg_base, nch)],
                            dst_v.at[pl.ds(0, nch)])
            for b in range(NBUF - 1):
                gather(b, b)

            def body(k2, _):
                j = NBUF * k2

                @pl.when(k2 > 0)
                def _():
                    swait(j - 1, NBUF - 1)

                gather(j + NBUF - 1, NBUF - 1)
                for b in range(NBUF - 1):
                    gwait(j + b, b)
                    scat(j + b, b)
                    swait(j + b, b)

                    @pl.when(k2 < nch // NBUF - 1)
                    def _():
                        gather(j + NBUF + b, b)

                gwait(j + NBUF - 1, NBUF - 1)
                scat(j + NBUF - 1, NBUF - 1)
                return 0

            lax.fori_loop(0, nch // NBUF, body, 0)
            swait(nch - 1, NBUF - 1)

        off = 0
        for nch in SEGS:
            run_seg(tbase + off, nch)
            off += nch

        @pl.when(has_extra)
        def _():
            run_seg(tbase + CBASE, CEXTRA)

        plsc.subcore_barrier()
        pltpu.sync_copy(acc_sh.at[pl.ds(s * RPT, RPT)],
                        out_hbm.at[c, pl.ds(s * RPT, RPT)])

    return k(hs, src_p, dst_p)


_BLK = 1000
_NBLK = N // _BLK


def _mm_kernel(x_ref, w_ref, o_ref):
    o_ref[...] = jnp.dot(x_ref[...], w_ref[...],
                         preferred_element_type=jnp.float32)


def _mm_bias_kernel(x_ref, w_ref, b_ref, o_ref):
    o_ref[...] = jnp.dot(x_ref[...], w_ref[...],
                         preferred_element_type=jnp.float32) + b_ref[...]


def _matmul(x, W, b=None):
    body = _mm_kernel if b is None else _mm_bias_kernel
    args = (x, W) if b is None else (x, W, b)
    specs = [
        pl.BlockSpec((_BLK, D), lambda i: (i, 0)),
        pl.BlockSpec((D, D), lambda i: (0, 0)),
    ]
    if b is not None:
        specs.append(pl.BlockSpec((1, D), lambda i: (0, 0)))
    return pl.pallas_call(
        body,
        grid=(_NBLK,),
        in_specs=specs,
        out_specs=pl.BlockSpec((_BLK, D), lambda i: (i, 0)),
        out_shape=jax.ShapeDtypeStruct((N, D), jnp.float32),
    )(*args)


def _scale_kernel(h_ref, degt_ref, hs_ref):
    d = degt_ref[...]
    deg = d[:, 0:1] + d[:, 1:2] + 1.0
    hs_ref[...] = h_ref[...] * lax.rsqrt(deg)


def _scale_by_dinv(h, degT):
    return pl.pallas_call(
        _scale_kernel,
        grid=(_NBLK,),
        in_specs=[
            pl.BlockSpec((_BLK, D), lambda i: (i, 0)),
            pl.BlockSpec((_BLK, NC), lambda i: (i, 0)),
        ],
        out_specs=pl.BlockSpec((_BLK, D), lambda i: (i, 0)),
        out_shape=jax.ShapeDtypeStruct((N, D), jnp.float32),
    )(h, degT)


def _bn_res_kernel(acc_ref, hs_ref, degt_ref, bc_ref, res_ref,
                   g_ref, b_ref, o_ref, t_sc, st_sc):
    i = pl.program_id(0)

    @pl.when(i < _NBLK)
    def _():
        d = degt_ref[...]
        deg = d[:, 0:1] + d[:, 1:2] + 1.0
        dinv = lax.rsqrt(deg)
        t = dinv * (acc_ref[0] + acc_ref[1] + hs_ref[...]) + bc_ref[...]
        t = jnp.maximum(t, 0.0)
        t_sc[pl.ds(i * _BLK, _BLK), :] = t

        @pl.when(i == 0)
        def _():
            st_sc[...] = jnp.zeros_like(st_sc)

        st_sc[0:1, :] += jnp.sum(t, axis=0, keepdims=True)
        st_sc[1:2, :] += jnp.sum(t * t, axis=0, keepdims=True)

    @pl.when(i >= _NBLK)
    def _():
        ii = i - _NBLK
        inv_n = 1.0 / N
        mean = st_sc[0:1, :] * inv_n
        var = st_sc[1:2, :] * inv_n - mean * mean
        scale = lax.rsqrt(var + EPS) * g_ref[...]
        t = t_sc[pl.ds(ii * _BLK, _BLK), :]
        o_ref[...] = (t - mean) * scale + b_ref[...] + res_ref[...]


def _compute_out(acc, hs, degT, b_conv2, res, gamma2, beta2):
    lo = lambda i: (jnp.minimum(i, _NBLK - 1),)
    hi = lambda i: (jnp.maximum(i - _NBLK, 0),)
    return pl.pallas_call(
        _bn_res_kernel,
        grid=(2 * _NBLK,),
        in_specs=[
            pl.BlockSpec((NC, _BLK, D), lambda i: (0,) + lo(i) + (0,)),
            pl.BlockSpec((_BLK, D), lambda i: lo(i) + (0,)),
            pl.BlockSpec((_BLK, NC), lambda i: lo(i) + (0,)),
            pl.BlockSpec((1, D), lambda i: (0, 0)),
            pl.BlockSpec((_BLK, D), lambda i: hi(i) + (0,)),
            pl.BlockSpec((1, D), lambda i: (0, 0)),
            pl.BlockSpec((1, D), lambda i: (0, 0)),
        ],
        out_specs=pl.BlockSpec((_BLK, D), lambda i: hi(i) + (0,)),
        out_shape=jax.ShapeDtypeStruct((N, D), jnp.float32),
        scratch_shapes=[
            pltpu.VMEM((N, D), jnp.float32),
            pltpu.VMEM((8, D), jnp.float32),
        ],
    )(acc, hs, degT, b_conv2, res, gamma2, beta2)


def kernel(x, edge_index, W_conv, b_conv, gamma, beta, W_res, b_res):
    dst_p = edge_index[1].astype(jnp.int32).reshape(NCHUNK, CH)
    ei2 = lax.optimization_barrier((edge_index, dst_p))[0]
    src_p = ei2[0].astype(jnp.int32).reshape(NCHUNK, CH)

    deg_parts = _deg_partials(dst_p)
    degT = deg_parts.T

    h = _matmul(x, W_conv)
    res = _matmul(x, W_res, b_res.reshape(1, D))
    hs = _scale_by_dinv(h, degT)
    acc = _scatter_partials(hs, src_p, dst_p)

    return _compute_out(acc, hs, degT, b_conv.reshape(1, D), res,
                        gamma.reshape(1, D), beta.reshape(1, D))


# speedup vs baseline: 1.1256x; 1.1256x over previous
import functools

import jax
import jax.numpy as jnp
from jax import lax
from jax.experimental import pallas as pl
from jax.experimental.pallas import tpu as pltpu
from jax.experimental.pallas import tpu_sc as plsc

N = 10000
D = 128
E = 320000
EPS = 1e-5
NC = 2
NS = 16
NW = NC * NS
CH = 128
NCHUNK = E // CH
CBASE = NCHUNK // NW
NEXTRA = NCHUNK - CBASE * NW
NACC = 10240
RPT = NACC // NS


def _tile_range(w):
    base = w * CBASE
    has_extra = w < NEXTRA
    return base, has_extra


def _mesh():
    return plsc.VectorSubcoreMesh(core_axis_name="c", subcore_axis_name="s")


def _deg_partials(ei3):
    @functools.partial(
        pl.kernel,
        out_type=jax.ShapeDtypeStruct((NC, NACC), jnp.float32),
        mesh=_mesh(),
        scratch_types=[
            pltpu.VMEM((CBASE + 1, 2, CH), jnp.int32),
            pltpu.VMEM((CH,), jnp.float32),
            pltpu.VMEM((RPT,), jnp.float32),
            pltpu.VMEM_SHARED((NACC,), jnp.float32),
            pltpu.SemaphoreType.DMA,
        ],
    )
    def k(ei_hbm, out_hbm, idx_v, ones_v, zeros_v, deg_sh, semd):
        c = lax.axis_index("c")
        s = lax.axis_index("s")
        w = s * NC + c
        base, has_extra = _tile_range(w)

        def fill_zeros(i, _):
            zeros_v[pl.ds(i * 16, 16)] = jnp.zeros((16,), jnp.float32)
            return 0

        lax.fori_loop(0, RPT // 16, fill_zeros, 0)

        def fill_ones(i, _):
            ones_v[pl.ds(i * 16, 16)] = jnp.ones((16,), jnp.float32)
            return 0

        lax.fori_loop(0, CH // 16, fill_ones, 0)

        pltpu.sync_copy(zeros_v, deg_sh.at[pl.ds(s * RPT, RPT)])
        plsc.subcore_barrier()

        pltpu.sync_copy(ei_hbm.at[pl.ds(base, CBASE)],
                        idx_v.at[pl.ds(0, CBASE)])

        @pl.when(has_extra)
        def _():
            pltpu.sync_copy(ei_hbm.at[pl.ds(NW * CBASE + w, 1)],
                            idx_v.at[pl.ds(CBASE, 1)])

        def body(j, _):
            pltpu.async_copy(ones_v, deg_sh.at[idx_v.at[j, 1]], semd,
                             add=True)
            return 0

        lax.fori_loop(0, CBASE, body, 0)

        def drain(j, _):
            pltpu.make_async_copy(ones_v, deg_sh.at[idx_v.at[0, 1]],
                                  semd).wait()
            return 0

        @pl.when(has_extra)
        def _():
            lax.fori_loop(CBASE, CBASE + 1, body, 0)
            lax.fori_loop(0, 1, drain, 0)

        lax.fori_loop(0, CBASE, drain, 0)
        plsc.subcore_barrier()
        pltpu.sync_copy(deg_sh.at[pl.ds(s * RPT, RPT)],
                        out_hbm.at[c, pl.ds(s * RPT, RPT)])

    return k(ei3)


SEG = 26
HCH = CH // 2


def _scatter_partials(hs, ei3):
    @functools.partial(
        pl.kernel,
        out_type=jax.ShapeDtypeStruct((NC, NACC, D), jnp.float32),
        mesh=_mesh(),
        scratch_types=[
            pltpu.VMEM((SEG + 1, 2, CH), jnp.int32),
            [pltpu.VMEM((CH, D), jnp.float32)] * 2,
            pltpu.VMEM_SHARED((NACC, D), jnp.float32),
            [pltpu.SemaphoreType.DMA] * 2,
            [pltpu.SemaphoreType.DMA] * 2,
        ],
    )
    def k(hs_hbm, ei_hbm, out_hbm, idx_v, rows, acc_sh, semg, sems):
        c = lax.axis_index("c")
        s = lax.axis_index("s")
        w = s * NC + c
        tbase, has_extra = _tile_range(w)

        def fill_zeros(t, _):
            rows[0][t // 8, pl.ds((t % 8) * 16, 16)] = jnp.zeros(
                (16,), jnp.float32)
            return 0

        lax.fori_loop(0, CH * 8, fill_zeros, 0)

        def zero_acc(i, _):
            pltpu.sync_copy(rows[0], acc_sh.at[pl.ds(s * RPT + i * CH, CH)])
            return 0

        lax.fori_loop(0, RPT // CH, zero_acc, 0)
        plsc.subcore_barrier()

        def gather(j, b):
            for h in range(2):
                pltpu.async_copy(
                    hs_hbm.at[idx_v.at[j, 0, pl.ds(h * HCH, HCH)]],
                    rows[b].at[pl.ds(h * HCH, HCH)], semg[b])

        def gwait(j, b):
            for h in range(2):
                pltpu.make_async_copy(
                    hs_hbm.at[idx_v.at[j, 0, pl.ds(h * HCH, HCH)]],
                    rows[b].at[pl.ds(h * HCH, HCH)], semg[b]).wait()

        def scat(j, b):
            pltpu.async_copy(rows[b], acc_sh.at[idx_v.at[j, 1]], sems[b],
                             add=True)

        def swait(j, b):
            pltpu.make_async_copy(rows[b], acc_sh.at[idx_v.at[j, 1]],
                                  sems[b]).wait()

        for seg in range(CBASE // SEG):
            base = tbase + seg * SEG
            pltpu.sync_copy(ei_hbm.at[pl.ds(base, SEG)],
                            idx_v.at[pl.ds(0, SEG)])
            gather(0, 0)

            def body(k2, _):
                j0 = 2 * k2
                j1 = j0 + 1

                @pl.when(k2 > 0)
                def _():
                    swait(j1 - 2, 1)

                gather(j1, 1)
                gwait(j0, 0)
                scat(j0, 0)
                swait(j0, 0)

                @pl.when(k2 < SEG // 2 - 1)
                def _():
                    gather(j0 + 2, 0)

                gwait(j1, 1)
                scat(j1, 1)
                return 0

            lax.fori_loop(0, SEG // 2, body, 0)
            swait(SEG - 1, 1)

        @pl.when(has_extra)
        def _():
            pltpu.sync_copy(ei_hbm.at[pl.ds(NW * CBASE + w, 1)],
                            idx_v.at[pl.ds(SEG, 1)])
            gather(SEG, 0)
            gwait(SEG, 0)
            scat(SEG, 0)
            swait(SEG, 0)

        plsc.subcore_barrier()
        pltpu.sync_copy(acc_sh.at[pl.ds(s * RPT, RPT)],
                        out_hbm.at[c, pl.ds(s * RPT, RPT)])

    return k(hs, ei3)


_BLK = 1000
_NBLK = N // _BLK


def _mm_kernel(x_ref, w_ref, o_ref):
    o_ref[...] = jnp.dot(x_ref[...], w_ref[...],
                         preferred_element_type=jnp.float32)


def _mm_bias_kernel(x_ref, w_ref, b_ref, o_ref):
    o_ref[...] = jnp.dot(x_ref[...], w_ref[...],
                         preferred_element_type=jnp.float32) + b_ref[...]


def _matmul(x, W, b=None):
    body = _mm_kernel if b is None else _mm_bias_kernel
    args = (x, W) if b is None else (x, W, b)
    specs = [
        pl.BlockSpec((_BLK, D), lambda i: (i, 0)),
        pl.BlockSpec((D, D), lambda i: (0, 0)),
    ]
    if b is not None:
        specs.append(pl.BlockSpec((1, D), lambda i: (0, 0)))
    return pl.pallas_call(
        body,
        grid=(_NBLK,),
        in_specs=specs,
        out_specs=pl.BlockSpec((_BLK, D), lambda i: (i, 0)),
        out_shape=jax.ShapeDtypeStruct((N, D), jnp.float32),
    )(*args)


def _hs_kernel(x_ref, w_ref, degt_ref, hs_ref):
    d = degt_ref[...]
    deg = d[:, 0:1] + d[:, 1:2] + 1.0
    dinv = lax.rsqrt(deg)
    h = jnp.dot(x_ref[...], w_ref[...], preferred_element_type=jnp.float32)
    hs_ref[...] = h * dinv


def _compute_hs(x, W_conv, degT):
    return pl.pallas_call(
        _hs_kernel,
        grid=(_NBLK,),
        in_specs=[
            pl.BlockSpec((_BLK, D), lambda i: (i, 0)),
            pl.BlockSpec((D, D), lambda i: (0, 0)),
            pl.BlockSpec((_BLK, NC), lambda i: (i, 0)),
        ],
        out_specs=pl.BlockSpec((_BLK, D), lambda i: (i, 0)),
        out_shape=jax.ShapeDtypeStruct((N, D), jnp.float32),
    )(x, W_conv, degT)


def _bn_res_kernel(acc_ref, hs_ref, degt_ref, bc_ref, res_ref,
                   g_ref, b_ref, o_ref, t_sc, st_sc):
    i = pl.program_id(0)

    @pl.when(i < _NBLK)
    def _():
        d = degt_ref[...]
        deg = d[:, 0:1] + d[:, 1:2] + 1.0
        dinv = lax.rsqrt(deg)
        t = dinv * (acc_ref[0] + acc_ref[1] + hs_ref[...]) + bc_ref[...]
        t = jnp.maximum(t, 0.0)
        t_sc[pl.ds(i * _BLK, _BLK), :] = t

        @pl.when(i == 0)
        def _():
            st_sc[...] = jnp.zeros_like(st_sc)

        st_sc[0:1, :] += jnp.sum(t, axis=0, keepdims=True)
        st_sc[1:2, :] += jnp.sum(t * t, axis=0, keepdims=True)

    @pl.when(i >= _NBLK)
    def _():
        ii = i - _NBLK
        inv_n = 1.0 / N
        mean = st_sc[0:1, :] * inv_n
        var = st_sc[1:2, :] * inv_n - mean * mean
        scale = lax.rsqrt(var + EPS) * g_ref[...]
        t = t_sc[pl.ds(ii * _BLK, _BLK), :]
        o_ref[...] = (t - mean) * scale + b_ref[...] + res_ref[...]


def _compute_out(acc, hs, degT, b_conv2, res, gamma2, beta2):
    lo = lambda i: (jnp.minimum(i, _NBLK - 1),)
    hi = lambda i: (jnp.maximum(i - _NBLK, 0),)
    return pl.pallas_call(
        _bn_res_kernel,
        grid=(2 * _NBLK,),
        in_specs=[
            pl.BlockSpec((NC, _BLK, D), lambda i: (0,) + lo(i) + (0,)),
            pl.BlockSpec((_BLK, D), lambda i: lo(i) + (0,)),
            pl.BlockSpec((_BLK, NC), lambda i: lo(i) + (0,)),
            pl.BlockSpec((1, D), lambda i: (0, 0)),
            pl.BlockSpec((_BLK, D), lambda i: hi(i) + (0,)),
            pl.BlockSpec((1, D), lambda i: (0, 0)),
            pl.BlockSpec((1, D), lambda i: (0, 0)),
        ],
        out_specs=pl.BlockSpec((_BLK, D), lambda i: hi(i) + (0,)),
        out_shape=jax.ShapeDtypeStruct((N, D), jnp.float32),
        scratch_shapes=[
            pltpu.VMEM((N, D), jnp.float32),
            pltpu.VMEM((8, D), jnp.float32),
        ],
    )(acc, hs, degT, b_conv2, res, gamma2, beta2)


def kernel(x, edge_index, W_conv, b_conv, gamma, beta, W_res, b_res):
    ei3 = (edge_index.astype(jnp.int32)
           .reshape(2, NCHUNK, CH).transpose(1, 0, 2))

    deg_parts = _deg_partials(ei3)
    degT = deg_parts.T

    hs = _compute_hs(x, W_conv, degT)
    res = _matmul(x, W_res, b_res.reshape(1, D))
    acc = _scatter_partials(hs, ei3)

    return _compute_out(acc, hs, degT, b_conv.reshape(1, D), res,
                        gamma.reshape(1, D), beta.reshape(1, D))
